# Initial kernel scaffold; baseline (speedup 1.0000x reference)
import jax
import jax.numpy as jnp
from jax.experimental import pallas as pl


def kernel(x, edge_index, W1l, W1r, b1, W2l, W2r, b2):
    raise NotImplementedError("write your pallas kernel here")


# speedup vs baseline: 3.9947x; 3.9947x over previous
import jax
import jax.numpy as jnp
from jax import lax
from jax.experimental import pallas as pl
from jax.experimental.pallas import tpu as pltpu
from jax.experimental.pallas import tpu_sc as plsc

N = 10000
E = 320000
D = 128
F = 64

NC = 2
NS = 16
K = 128
CH = 16
NBT = -(-E // (NS * K * CH)) * CH
E_PAD = NS * NBT * K
ROWS_PER_TILE = 640
N_PAD = NS * ROWS_PER_TILE

_f32 = jnp.float32


def _sc_scatter_body(with_deg, y_hbm, src_hbm, dst_hbm, acc_out, deg_out,
                     src_v, dst_v, rows_v, ones_v, zbuf16, acc_sh, deg_sh,
                     sem):
    cid = lax.axis_index("c")
    sid = lax.axis_index("s")

    def _zrow(i, _):
        def _zcol(j, _):
            rows_v[i, pl.ds(j * 16, 16)] = jnp.zeros((16,), _f32)
            return 0
        return lax.fori_loop(0, F // 16, _zcol, 0)
    lax.fori_loop(0, K, _zrow, 0)

    def _zchunk(t, _):
        pltpu.sync_copy(rows_v, acc_sh.at[pl.ds(sid * ROWS_PER_TILE + t * K, K)])
        return 0
    lax.fori_loop(0, ROWS_PER_TILE // K, _zchunk, 0)

    if with_deg:
        def _zrow16(i, _):
            zbuf16[i, pl.ds(0, 16)] = jnp.zeros((16,), _f32)
            return 0
        lax.fori_loop(0, 64, _zrow16, 0)

        def _zchunk16(t, _):
            pltpu.sync_copy(zbuf16,
                            deg_sh.at[pl.ds(sid * ROWS_PER_TILE + t * 64, 64)])
            return 0
        lax.fori_loop(0, ROWS_PER_TILE // 64, _zchunk16, 0)

        def _onesrow(i, _):
            ones_v[i, pl.ds(0, 16)] = jnp.ones((16,), _f32)
            return 0
        lax.fori_loop(0, K, _onesrow, 0)

    plsc.subcore_barrier()

    ytab = y_hbm.at[cid]

    def _chunk(t, _):
        base = sid * NBT + t * CH
        pltpu.sync_copy(src_hbm.at[pl.ds(base, CH)], src_v)
        pltpu.sync_copy(dst_hbm.at[pl.ds(base, CH)], dst_v)

        def _blk(jj, _):
            pltpu.async_copy(ytab.at[src_v.at[jj]], rows_v, sem).wait()
            pltpu.sync_copy(rows_v, acc_sh.at[dst_v.at[jj]], add=True)
            if with_deg:
                @pl.when((t * CH + jj) // (NBT // 2) == cid)
                def _():
                    pltpu.sync_copy(ones_v, deg_sh.at[dst_v.at[jj]], add=True)
            return 0
        return lax.fori_loop(0, CH, _blk, 0)
    lax.fori_loop(0, NBT // CH, _chunk, 0)

    plsc.subcore_barrier()

    base = sid * ROWS_PER_TILE
    pltpu.sync_copy(acc_sh.at[pl.ds(base, ROWS_PER_TILE)],
                    acc_out.at[cid, pl.ds(base, ROWS_PER_TILE)])
    if with_deg:
        pltpu.sync_copy(deg_sh.at[pl.ds(base, ROWS_PER_TILE)],
                        deg_out.at[cid, pl.ds(base, ROWS_PER_TILE)])


def _make_sc_scatter(with_deg):
    mesh = plsc.VectorSubcoreMesh(core_axis_name="c", subcore_axis_name="s",
                                  num_cores=NC, num_subcores=NS)
    out_type = [jax.ShapeDtypeStruct((NC, N_PAD, F), _f32)]
    if with_deg:
        out_type.append(jax.ShapeDtypeStruct((NC, N_PAD, 16), _f32))
    scratch = [
        pltpu.VMEM((CH, K), jnp.int32),
        pltpu.VMEM((CH, K), jnp.int32),
        pltpu.VMEM((K, F), _f32),
        pltpu.VMEM((K, 16), _f32),
        pltpu.VMEM((64, 16), _f32),
        pltpu.VMEM_SHARED((N_PAD, F), _f32),
        pltpu.VMEM_SHARED((N_PAD, 16), _f32),
        pltpu.SemaphoreType.DMA,
    ]
    if not with_deg:
        scratch = [scratch[0], scratch[1], scratch[2], scratch[5], scratch[7]]

    def body(y_hbm, src_hbm, dst_hbm, *rest):
        if with_deg:
            acc_out, deg_out = rest[0], rest[1]
            src_v, dst_v, rows_v, ones_v, zbuf16, acc_sh, deg_sh, sem = rest[2:]
        else:
            acc_out, deg_out = rest[0], None
            src_v, dst_v, rows_v, acc_sh, sem = rest[1:]
            ones_v = zbuf16 = deg_sh = None
        _sc_scatter_body(with_deg, y_hbm, src_hbm, dst_hbm, acc_out, deg_out,
                         src_v, dst_v, rows_v, ones_v, zbuf16, acc_sh, deg_sh,
                         sem)

    return pl.kernel(body, out_type=tuple(out_type), mesh=mesh,
                     scratch_types=scratch,
                     compiler_params=pltpu.CompilerParams(
                         use_tc_tiling_on_sc=False))


_sc_scatter_deg = _make_sc_scatter(True)
_sc_scatter = _make_sc_scatter(False)


BR = 1000


def _tc1_body(x_ref, wl_ref, wr_ref, b_ref, ylo_ref, yhi_ref, z_ref):
    x = x_ref[...]
    y = jnp.dot(x, wl_ref[...], preferred_element_type=_f32)
    ylo_ref[...] = y[:, :F]
    yhi_ref[...] = y[:, F:]
    z_ref[...] = jnp.dot(x, wr_ref[...], preferred_element_type=_f32) + b_ref[...]


def _tc1(x, wl, wr, b):
    return pl.pallas_call(
        _tc1_body,
        grid=(N // BR,),
        in_specs=[
            pl.BlockSpec((BR, D), lambda i: (i, 0)),
            pl.BlockSpec((D, D), lambda i: (0, 0)),
            pl.BlockSpec((D, D), lambda i: (0, 0)),
            pl.BlockSpec((1, D), lambda i: (0, 0)),
        ],
        out_specs=[
            pl.BlockSpec((BR, F), lambda i: (i, 0)),
            pl.BlockSpec((BR, F), lambda i: (i, 0)),
            pl.BlockSpec((BR, D), lambda i: (i, 0)),
        ],
        out_shape=[jax.ShapeDtypeStruct((N, F), _f32),
                   jax.ShapeDtypeStruct((N, F), _f32),
                   jax.ShapeDtypeStruct((N, D), _f32)],
    )(x, wl, wr, b)


def _combine(acc_ref, dp_ref, z_ref):
    a = acc_ref[...]
    s = jnp.concatenate([a[0], a[1]], axis=-1)
    deg = dp_ref[0, :, 0] + dp_ref[1, :, 0]
    inv = 1.0 / jnp.maximum(deg, 1.0)
    return s * inv[:, None] + z_ref[...]


def _tc2_body(acc_ref, dp_ref, z1_ref, wl_ref, wr_ref, b_ref,
              y2lo_ref, y2hi_ref, z2_ref):
    h = jnp.maximum(_combine(acc_ref, dp_ref, z1_ref), 0.0)
    y2 = jnp.dot(h, wl_ref[...], preferred_element_type=_f32)
    y2lo_ref[...] = y2[:, :F]
    y2hi_ref[...] = y2[:, F:]
    z2_ref[...] = jnp.dot(h, wr_ref[...], preferred_element_type=_f32) + b_ref[...]


def _tc2(acc, dp, z1, wl, wr, b):
    return pl.pallas_call(
        _tc2_body,
        grid=(N // BR,),
        in_specs=[
            pl.BlockSpec((NC, BR, F), lambda i: (0, i, 0)),
            pl.BlockSpec((NC, BR, 16), lambda i: (0, i, 0)),
            pl.BlockSpec((BR, D), lambda i: (i, 0)),
            pl.BlockSpec((D, D), lambda i: (0, 0)),
            pl.BlockSpec((D, D), lambda i: (0, 0)),
            pl.BlockSpec((1, D), lambda i: (0, 0)),
        ],
        out_specs=[
            pl.BlockSpec((BR, F), lambda i: (i, 0)),
            pl.BlockSpec((BR, F), lambda i: (i, 0)),
            pl.BlockSpec((BR, D), lambda i: (i, 0)),
        ],
        out_shape=[jax.ShapeDtypeStruct((N, F), _f32),
                   jax.ShapeDtypeStruct((N, F), _f32),
                   jax.ShapeDtypeStruct((N, D), _f32)],
    )(acc, dp, z1, wl, wr, b)


def _tc3_body(acc_ref, dp_ref, z2_ref, o_ref):
    o_ref[...] = _combine(acc_ref, dp_ref, z2_ref)


def _tc3(acc, dp, z2):
    return pl.pallas_call(
        _tc3_body,
        grid=(N // BR,),
        in_specs=[
            pl.BlockSpec((NC, BR, F), lambda i: (0, i, 0)),
            pl.BlockSpec((NC, BR, 16), lambda i: (0, i, 0)),
            pl.BlockSpec((BR, D), lambda i: (i, 0)),
        ],
        out_specs=pl.BlockSpec((BR, D), lambda i: (i, 0)),
        out_shape=jax.ShapeDtypeStruct((N, D), _f32),
    )(acc, dp, z2)


def kernel(x, edge_index, W1l, W1r, b1, W2l, W2r, b2):
    src = edge_index[0]
    dst = edge_index[1]
    pad = E_PAD - E
    src_p = jnp.concatenate([src, jnp.zeros((pad,), jnp.int32)])
    dst_p = jnp.concatenate([dst, jnp.full((pad,), N, jnp.int32)])
    src2d = src_p.reshape(NS * NBT, K)
    dst2d = dst_p.reshape(NS * NBT, K)

    b1r = b1.reshape(1, D)
    b2r = b2.reshape(1, D)

    y1lo, y1hi, z1 = _tc1(x, W1l, W1r, b1r)
    y1s = jnp.stack([y1lo, y1hi])
    acc1, degp = _sc_scatter_deg(y1s, src2d, dst2d)
    y2lo, y2hi, z2 = _tc2(acc1, degp, z1, W2l, W2r, b2r)
    y2s = jnp.stack([y2lo, y2hi])
    acc2, = _sc_scatter(y2s, src2d, dst2d)
    return _tc3(acc2, degp, z2)


# speedup vs baseline: 5.4905x; 1.3745x over previous
import jax
import jax.numpy as jnp
from jax import lax
from jax.experimental import pallas as pl
from jax.experimental.pallas import tpu as pltpu
from jax.experimental.pallas import tpu_sc as plsc

N = 10000
E = 320000
D = 128
F = 64

NC = 2
NS = 16
K = 128
CH = 40
M = 5
G = 3
NBT = -(-E // (NS * K * 40)) * 40
E_PAD = NS * NBT * K
ROWS_PER_TILE = 640
N_PAD = NS * ROWS_PER_TILE

_f32 = jnp.float32


def _sc_scatter_body(with_deg, y_hbm, src_hbm, dst_hbm, acc_out, deg_out,
                     src_v, dst_v, rows_v, ones_v, zbuf16, acc_sh, deg_sh,
                     sem_g, sem_d, sem_i, sem_s):
    cid = lax.axis_index("c")
    sid = lax.axis_index("s")

    def _zrow(i, _):
        def _zcol(j, _):
            rows_v[0, i, pl.ds(j * 16, 16)] = jnp.zeros((16,), _f32)
            return 0
        return lax.fori_loop(0, F // 16, _zcol, 0)
    lax.fori_loop(0, K, _zrow, 0)

    def _zchunk(t, _):
        pltpu.sync_copy(rows_v.at[0],
                        acc_sh.at[pl.ds(sid * ROWS_PER_TILE + t * K, K)])
        return 0
    lax.fori_loop(0, ROWS_PER_TILE // K, _zchunk, 0)

    if with_deg:
        def _zrow16(i, _):
            zbuf16[i, pl.ds(0, 16)] = jnp.zeros((16,), _f32)
            return 0
        lax.fori_loop(0, 64, _zrow16, 0)

        def _zchunk16(t, _):
            pltpu.sync_copy(zbuf16,
                            deg_sh.at[pl.ds(sid * ROWS_PER_TILE + t * 64, 64)])
            return 0
        lax.fori_loop(0, ROWS_PER_TILE // 64, _zchunk16, 0)

        def _onesrow(i, _):
            ones_v[i, pl.ds(0, 16)] = jnp.ones((16,), _f32)
            return 0
        lax.fori_loop(0, K, _onesrow, 0)

    plsc.subcore_barrier()

    ytab = y_hbm.at[cid]
    NCH = NBT // CH

    pltpu.sync_copy(src_hbm.at[pl.ds(sid * NBT, CH)], src_v.at[0])
    pltpu.sync_copy(dst_hbm.at[pl.ds(sid * NBT, CH)], dst_v.at[0])

    def _chunk(t, _):
        tb = t % 2
        sv = src_v.at[tb]
        dv = dst_v.at[tb]

        @pl.when(t + 1 < NCH)
        def _():
            b2 = sid * NBT + (t + 1) * CH
            pltpu.async_copy(src_hbm.at[pl.ds(b2, CH)], src_v.at[1 - tb],
                             sem_i)
            pltpu.async_copy(dst_hbm.at[pl.ds(b2, CH)], dst_v.at[1 - tb],
                             sem_i)

        gd = [None] * CH
        sd = [None] * CH
        for p in range(G):
            @pl.when(t > 0)
            def _(b=p % M):
                pltpu.make_async_copy(ytab.at[pl.ds(0, K)], rows_v.at[b],
                                      sem_s[b]).wait()
            gd[p] = pltpu.async_copy(ytab.at[sv.at[p]], rows_v.at[p % M],
                                     sem_g)
        for jj in range(CH):
            nxt = jj + G
            if nxt < CH:
                prev = nxt - M
                if prev >= 0:
                    sd[prev].wait()
                else:
                    @pl.when(t > 0)
                    def _(b=nxt % M):
                        pltpu.make_async_copy(ytab.at[pl.ds(0, K)],
                                              rows_v.at[b], sem_s[b]).wait()
                gd[nxt] = pltpu.async_copy(ytab.at[sv.at[nxt]],
                                           rows_v.at[nxt % M], sem_g)
            gd[jj].wait()
            sd[jj] = pltpu.async_copy(rows_v.at[jj % M],
                                      acc_sh.at[dv.at[jj]],
                                      sem_s[jj % M], add=True)
        if with_deg:
            for parity in (0, 1):
                @pl.when(cid == parity)
                def _(parity=parity):
                    dd = [pltpu.async_copy(ones_v, deg_sh.at[dv.at[jj]],
                                           sem_d, add=True)
                          for jj in range(parity, CH, 2)]
                    for dsc in dd:
                        dsc.wait()

        @pl.when(t + 1 < NCH)
        def _():
            b2 = sid * NBT + (t + 1) * CH
            pltpu.make_async_copy(src_hbm.at[pl.ds(b2, CH)],
                                  src_v.at[1 - tb], sem_i).wait()
            pltpu.make_async_copy(dst_hbm.at[pl.ds(b2, CH)],
                                  dst_v.at[1 - tb], sem_i).wait()
        return 0
    lax.fori_loop(0, NCH, _chunk, 0)

    for b in range(M):
        pltpu.make_async_copy(ytab.at[pl.ds(0, K)], rows_v.at[b],
                              sem_s[b]).wait()

    plsc.subcore_barrier()

    base = sid * ROWS_PER_TILE
    pltpu.sync_copy(acc_sh.at[pl.ds(base, ROWS_PER_TILE)],
                    acc_out.at[cid, pl.ds(base, ROWS_PER_TILE)])
    if with_deg:
        pltpu.sync_copy(deg_sh.at[pl.ds(base, ROWS_PER_TILE)],
                        deg_out.at[cid, pl.ds(base, ROWS_PER_TILE)])


def _make_sc_scatter(with_deg):
    mesh = plsc.VectorSubcoreMesh(core_axis_name="c", subcore_axis_name="s",
                                  num_cores=NC, num_subcores=NS)
    out_type = [jax.ShapeDtypeStruct((NC, N_PAD, F), _f32)]
    if with_deg:
        out_type.append(jax.ShapeDtypeStruct((NC, N_PAD, 16), _f32))
    scratch = [
        pltpu.VMEM((2, CH, K), jnp.int32),
        pltpu.VMEM((2, CH, K), jnp.int32),
        pltpu.VMEM((M, K, F), _f32),
        pltpu.VMEM((K, 16), _f32),
        pltpu.VMEM((64, 16), _f32),
        pltpu.VMEM_SHARED((N_PAD, F), _f32),
        pltpu.VMEM_SHARED((N_PAD, 16), _f32),
        pltpu.SemaphoreType.DMA,
        pltpu.SemaphoreType.DMA,
        pltpu.SemaphoreType.DMA,
    ] + [pltpu.SemaphoreType.DMA] * M
    if not with_deg:
        scratch = scratch[:3] + scratch[5:6] + scratch[7:]

    def body(y_hbm, src_hbm, dst_hbm, *rest):
        if with_deg:
            acc_out, deg_out = rest[0], rest[1]
            (src_v, dst_v, rows_v, ones_v, zbuf16, acc_sh, deg_sh,
             sem_g, sem_d, sem_i) = rest[2:12]
            sem_s = list(rest[12:])
        else:
            acc_out, deg_out = rest[0], None
            src_v, dst_v, rows_v, acc_sh, sem_g, sem_d, sem_i = rest[1:8]
            sem_s = list(rest[8:])
            ones_v = zbuf16 = deg_sh = None
        _sc_scatter_body(with_deg, y_hbm, src_hbm, dst_hbm, acc_out, deg_out,
                         src_v, dst_v, rows_v, ones_v, zbuf16, acc_sh, deg_sh,
                         sem_g, sem_d, sem_i, sem_s)

    return pl.kernel(body, out_type=tuple(out_type), mesh=mesh,
                     scratch_types=scratch,
                     compiler_params=pltpu.CompilerParams(
                         use_tc_tiling_on_sc=False))


_sc_scatter_deg = _make_sc_scatter(True)
_sc_scatter = _make_sc_scatter(False)


BR = 1000


def _tc1_body(x_ref, wl_ref, wr_ref, b_ref, ylo_ref, yhi_ref, z_ref):
    x = x_ref[...]
    y = jnp.dot(x, wl_ref[...], preferred_element_type=_f32)
    ylo_ref[...] = y[:, :F]
    yhi_ref[...] = y[:, F:]
    z_ref[...] = jnp.dot(x, wr_ref[...], preferred_element_type=_f32) + b_ref[...]


def _tc1(x, wl, wr, b):
    return pl.pallas_call(
        _tc1_body,
        grid=(N // BR,),
        in_specs=[
            pl.BlockSpec((BR, D), lambda i: (i, 0)),
            pl.BlockSpec((D, D), lambda i: (0, 0)),
            pl.BlockSpec((D, D), lambda i: (0, 0)),
            pl.BlockSpec((1, D), lambda i: (0, 0)),
        ],
        out_specs=[
            pl.BlockSpec((BR, F), lambda i: (i, 0)),
            pl.BlockSpec((BR, F), lambda i: (i, 0)),
            pl.BlockSpec((BR, D), lambda i: (i, 0)),
        ],
        out_shape=[jax.ShapeDtypeStruct((N, F), _f32),
                   jax.ShapeDtypeStruct((N, F), _f32),
                   jax.ShapeDtypeStruct((N, D), _f32)],
    )(x, wl, wr, b)


def _combine(acc_ref, dp_ref, z_ref):
    a = acc_ref[...]
    s = jnp.concatenate([a[0], a[1]], axis=-1)
    deg = dp_ref[0, :, 0] + dp_ref[1, :, 0]
    inv = 1.0 / jnp.maximum(deg, 1.0)
    return s * inv[:, None] + z_ref[...]


def _tc2_body(acc_ref, dp_ref, z1_ref, wl_ref, wr_ref, b_ref,
              y2lo_ref, y2hi_ref, z2_ref):
    h = jnp.maximum(_combine(acc_ref, dp_ref, z1_ref), 0.0)
    y2 = jnp.dot(h, wl_ref[...], preferred_element_type=_f32)
    y2lo_ref[...] = y2[:, :F]
    y2hi_ref[...] = y2[:, F:]
    z2_ref[...] = jnp.dot(h, wr_ref[...], preferred_element_type=_f32) + b_ref[...]


def _tc2(acc, dp, z1, wl, wr, b):
    return pl.pallas_call(
        _tc2_body,
        grid=(N // BR,),
        in_specs=[
            pl.BlockSpec((NC, BR, F), lambda i: (0, i, 0)),
            pl.BlockSpec((NC, BR, 16), lambda i: (0, i, 0)),
            pl.BlockSpec((BR, D), lambda i: (i, 0)),
            pl.BlockSpec((D, D), lambda i: (0, 0)),
            pl.BlockSpec((D, D), lambda i: (0, 0)),
            pl.BlockSpec((1, D), lambda i: (0, 0)),
        ],
        out_specs=[
            pl.BlockSpec((BR, F), lambda i: (i, 0)),
            pl.BlockSpec((BR, F), lambda i: (i, 0)),
            pl.BlockSpec((BR, D), lambda i: (i, 0)),
        ],
        out_shape=[jax.ShapeDtypeStruct((N, F), _f32),
                   jax.ShapeDtypeStruct((N, F), _f32),
                   jax.ShapeDtypeStruct((N, D), _f32)],
    )(acc, dp, z1, wl, wr, b)


def _tc3_body(acc_ref, dp_ref, z2_ref, o_ref):
    o_ref[...] = _combine(acc_ref, dp_ref, z2_ref)


def _tc3(acc, dp, z2):
    return pl.pallas_call(
        _tc3_body,
        grid=(N // BR,),
        in_specs=[
            pl.BlockSpec((NC, BR, F), lambda i: (0, i, 0)),
            pl.BlockSpec((NC, BR, 16), lambda i: (0, i, 0)),
            pl.BlockSpec((BR, D), lambda i: (i, 0)),
        ],
        out_specs=pl.BlockSpec((BR, D), lambda i: (i, 0)),
        out_shape=jax.ShapeDtypeStruct((N, D), _f32),
    )(acc, dp, z2)


def kernel(x, edge_index, W1l, W1r, b1, W2l, W2r, b2):
    src = edge_index[0]
    dst = edge_index[1]
    pad = E_PAD - E
    src_p = jnp.concatenate([src, jnp.zeros((pad,), jnp.int32)])
    dst_p = jnp.concatenate([dst, jnp.full((pad,), N, jnp.int32)])
    src2d = src_p.reshape(NS * NBT, K)
    dst2d = dst_p.reshape(NS * NBT, K)

    b1r = b1.reshape(1, D)
    b2r = b2.reshape(1, D)

    y1lo, y1hi, z1 = _tc1(x, W1l, W1r, b1r)
    y1s = jnp.stack([y1lo, y1hi])
    acc1, degp = _sc_scatter_deg(y1s, src2d, dst2d)
    y2lo, y2hi, z2 = _tc2(acc1, degp, z1, W2l, W2r, b2r)
    y2s = jnp.stack([y2lo, y2hi])
    acc2, = _sc_scatter(y2s, src2d, dst2d)
    return _tc3(acc2, degp, z2)
